# baseline (speedup 1.0000x reference)
import functools

import jax
import jax.numpy as jnp
from jax import lax
from jax.experimental import pallas as pl
from jax.experimental.pallas import tpu as pltpu
from jax.experimental.pallas import tpu_sc as plsc

_EPS = 1e-5
_TOPK = 4


def _rms_router_body(x_ref, rw_ref, w_ref, b_ref, h_ref, logits_ref):
    x = x_ref[...]
    var = jnp.mean(x * x, axis=-1, keepdims=True)
    h = x * jax.lax.rsqrt(var + _EPS) * rw_ref[...]
    logits_ref[...] = jax.lax.dot_general(
        h, w_ref[...], (((1,), (1,)), ((), ())),
        preferred_element_type=jnp.float32) + b_ref[...]
    h_ref[...] = h


def _make_sc_router(T, E):
    info = plsc.get_sparse_core_info()
    NC, NS = info.num_cores, info.num_subcores
    NW = NC * NS
    rows_per_w = T // NW
    mesh = plsc.VectorSubcoreMesh(core_axis_name="c", subcore_axis_name="s")

    @functools.partial(
        pl.kernel,
        out_type=jax.ShapeDtypeStruct((T, E), jnp.float32),
        mesh=mesh,
        scratch_types=[
            pltpu.VMEM((rows_per_w, E), jnp.float32),
            pltpu.VMEM((rows_per_w, E), jnp.float32),
        ],
        compiler_params=pltpu.CompilerParams(needs_layout_passes=False),
    )
    def _sc_router(logits_hbm, aff_hbm, rows_v, out_v):
        wid = lax.axis_index("s") * NC + lax.axis_index("c")
        base = wid * rows_per_w
        pltpu.sync_copy(logits_hbm.at[pl.ds(base, rows_per_w)], rows_v)
        iota = lax.iota(jnp.int32, E)

        def _bcast_max(v):
            return plsc.cummax(lax.rev(plsc.cummax(v), (0,)))

        for r in range(rows_per_w):
            row = rows_v[r]
            avail = jnp.ones((E,), jnp.bool_)
            sel_any = jnp.zeros((E,), jnp.bool_)
            m0 = None
            for _ in range(_TOPK):
                cur = jnp.where(avail, row, jnp.float32(-1e30))
                m = _bcast_max(cur)
                if m0 is None:
                    m0 = m
                is_m = (cur == m) & avail
                negidx = jnp.where(is_m, -iota, jnp.int32(-E))
                b = _bcast_max(negidx)
                sel = is_m & (negidx == b)
                avail = avail & jnp.logical_not(sel)
                sel_any = sel_any | sel
            expd = jnp.exp(row - m0)
            masked = jnp.where(sel_any, expd, jnp.float32(0.0))
            z = plsc.cummax(lax.rev(plsc.cumsum(masked), (0,)))
            out_v[r] = masked / z
        pltpu.sync_copy(out_v, aff_hbm.at[pl.ds(base, rows_per_w)])

    return _sc_router


def _topk_affinities(logits):
    T, E = logits.shape
    iota = jax.lax.broadcasted_iota(jnp.int32, (T, E), 1)
    avail = jnp.ones((T, E), dtype=jnp.bool_)
    neg = jnp.float32(-1e30)
    sels, ms = [], []
    for _ in range(_TOPK):
        cur = jnp.where(avail, logits, neg)
        m = jnp.max(cur, axis=-1, keepdims=True)
        is_m = (cur == m) & avail
        fi = jnp.min(jnp.where(is_m, iota, E), axis=-1, keepdims=True)
        sel = iota == fi
        avail = avail & jnp.logical_not(sel)
        sels.append(sel)
        ms.append(m)
    m0 = ms[0]
    es = [jnp.exp(m - m0) for m in ms]
    z = es[0]
    for t in es[1:]:
        z = z + t
    aff = jnp.zeros((T, E), jnp.float32)
    for sel, t in zip(sels, es):
        aff = aff + sel.astype(jnp.float32) * (t / z)
    return aff


def _make_pipe_body(E_, NH, FB, I):
    def _body(h_ref, logits_ref, x_ref, wgu_ref, bgu_ref, wd_ref, bd_ref,
              out_ref, aff_ref, gu0_ref, gu1_ref):
        e = pl.program_id(0)
        s = pl.program_id(1)
        par = jax.lax.rem(e, 2)

        @pl.when((e == 0) & (s == 0))
        def _init():
            aff_ref[...] = _topk_affinities(logits_ref[...])
            out_ref[...] = x_ref[...] + jnp.dot(
                aff_ref[...], bd_ref[...], preferred_element_type=jnp.float32)

        def _acc(gu):
            @pl.when(s == 0)
            def _a0():
                gu[...] = jnp.dot(
                    h_ref[...], wgu_ref[0],
                    preferred_element_type=jnp.float32) + bgu_ref[0]

            @pl.when(s > 0)
            def _a1():
                gu[...] += jnp.dot(
                    h_ref[...], wgu_ref[0],
                    preferred_element_type=jnp.float32)

        @pl.when((e < E_) & (par == 0))
        def _acc0():
            _acc(gu0_ref)

        @pl.when((e < E_) & (par == 1))
        def _acc1():
            _acc(gu1_ref)

        def _down(gu):
            gate = gu[:, pl.ds(s * FB, FB)]
            up = gu[:, pl.ds(I + s * FB, FB)]
            inter = up * jax.nn.sigmoid(gate)
            contrib = jnp.dot(inter, wd_ref[0],
                              preferred_element_type=jnp.float32)
            onehot = (jax.lax.broadcasted_iota(jnp.int32, (E_, 1), 0)
                      == (e - 1)).astype(jnp.float32)
            aff_col = jnp.dot(aff_ref[...], onehot,
                              preferred_element_type=jnp.float32)
            out_ref[...] += aff_col * contrib

        @pl.when((e > 0) & (par == 1))
        def _down0():
            _down(gu0_ref)

        @pl.when((e > 0) & (par == 0))
        def _down1():
            _down(gu1_ref)

    return _body


def kernel(hidden_states, rms_weight, router_weight, router_bias,
           W_gu, b_gu, W_down, b_down):
    T, H = hidden_states.shape
    E, _, I2 = W_gu.shape
    I = I2 // 2
    FB = 512
    NF = I // FB

    h, logits = pl.pallas_call(
        _rms_router_body,
        out_shape=(
            jax.ShapeDtypeStruct((T, H), jnp.float32),
            jax.ShapeDtypeStruct((T, E), jnp.float32),
        ),
    )(hidden_states, rms_weight.reshape(1, H), router_weight,
      router_bias.reshape(1, E))

    aff = _make_sc_router(T, E)(logits)

    HB = H // 6
    NH = H // HB
    FBP = I // NH

    out = pl.pallas_call(
        _make_pipe_body(E, NH, FBP, I),
        grid=(E + 1, NH),
        in_specs=[
            pl.BlockSpec((T, HB),
                         lambda e, s: (0, jnp.where(e >= E, NH - 1, s))),
            pl.BlockSpec((T, E), lambda e, s: (0, 0)),
            pl.BlockSpec((T, H), lambda e, s: (0, 0)),
            pl.BlockSpec((1, HB, I2),
                         lambda e, s: (jnp.minimum(e, E - 1),
                                       jnp.where(e >= E, NH - 1, s), 0)),
            pl.BlockSpec((1, 1, I2),
                         lambda e, s: (jnp.minimum(e, E - 1), 0, 0)),
            pl.BlockSpec((1, FBP, H),
                         lambda e, s: (jnp.maximum(e - 1, 0),
                                       jnp.where(e == 0, 0, s), 0)),
            pl.BlockSpec((E, H), lambda e, s: (0, 0)),
        ],
        out_specs=pl.BlockSpec((T, H), lambda e, s: (0, 0)),
        out_shape=jax.ShapeDtypeStruct((T, H), jnp.float32),
        scratch_shapes=[
            pltpu.VMEM((T, E), jnp.float32),
            pltpu.VMEM((T, I2), jnp.float32),
            pltpu.VMEM((T, I2), jnp.float32),
        ],
        compiler_params=pltpu.CompilerParams(
            vmem_limit_bytes=100 * 1024 * 1024),
    )(h, logits, hidden_states, W_gu, b_gu.reshape(E, 1, I2), W_down, b_down)

    return out, aff


# speedup vs baseline: 1.0261x; 1.0261x over previous
import functools

import jax
import jax.numpy as jnp
from jax import lax
from jax.experimental import pallas as pl
from jax.experimental.pallas import tpu as pltpu
from jax.experimental.pallas import tpu_sc as plsc

_EPS = 1e-5
_TOPK = 4


def _rms_router_body(x_ref, rw_ref, w_ref, b_ref, h_ref, logits_ref):
    x = x_ref[...]
    var = jnp.mean(x * x, axis=-1, keepdims=True)
    h = x * jax.lax.rsqrt(var + _EPS) * rw_ref[...]
    logits_ref[...] = jax.lax.dot_general(
        h, w_ref[...], (((1,), (1,)), ((), ())),
        preferred_element_type=jnp.float32) + b_ref[...]
    h_ref[...] = h


def _make_sc_router(T, E):
    info = plsc.get_sparse_core_info()
    NC, NS = 1, info.num_subcores
    NW = NC * NS
    rows_per_w = T // NW
    mesh = plsc.VectorSubcoreMesh(core_axis_name="c", subcore_axis_name="s",
                                  num_cores=NC)

    @functools.partial(
        pl.kernel,
        out_type=jax.ShapeDtypeStruct((T, E), jnp.float32),
        mesh=mesh,
        scratch_types=[
            pltpu.VMEM((rows_per_w, E), jnp.float32),
            pltpu.VMEM((rows_per_w, E), jnp.float32),
        ],
        compiler_params=pltpu.CompilerParams(needs_layout_passes=False),
    )
    def _sc_router(logits_hbm, aff_hbm, rows_v, out_v):
        wid = lax.axis_index("s") * NC + lax.axis_index("c")
        base = wid * rows_per_w
        pltpu.sync_copy(logits_hbm.at[pl.ds(base, rows_per_w)], rows_v)
        iota = lax.iota(jnp.int32, E)

        def _bcast_max(v):
            return plsc.cummax(lax.rev(plsc.cummax(v), (0,)))

        for r in range(rows_per_w):
            row = rows_v[r]
            avail = jnp.ones((E,), jnp.bool_)
            sel_any = jnp.zeros((E,), jnp.bool_)
            m0 = None
            for _ in range(_TOPK):
                cur = jnp.where(avail, row, jnp.float32(-1e30))
                m = _bcast_max(cur)
                if m0 is None:
                    m0 = m
                is_m = (cur == m) & avail
                negidx = jnp.where(is_m, -iota, jnp.int32(-E))
                b = _bcast_max(negidx)
                sel = is_m & (negidx == b)
                avail = avail & jnp.logical_not(sel)
                sel_any = sel_any | sel
            expd = jnp.exp(row - m0)
            masked = jnp.where(sel_any, expd, jnp.float32(0.0))
            z = plsc.cummax(lax.rev(plsc.cumsum(masked), (0,)))
            out_v[r] = masked / z
        pltpu.sync_copy(out_v, aff_hbm.at[pl.ds(base, rows_per_w)])

    return _sc_router


def _topk_affinities(logits):
    T, E = logits.shape
    iota = jax.lax.broadcasted_iota(jnp.int32, (T, E), 1)
    avail = jnp.ones((T, E), dtype=jnp.bool_)
    neg = jnp.float32(-1e30)
    sels, ms = [], []
    for _ in range(_TOPK):
        cur = jnp.where(avail, logits, neg)
        m = jnp.max(cur, axis=-1, keepdims=True)
        is_m = (cur == m) & avail
        fi = jnp.min(jnp.where(is_m, iota, E), axis=-1, keepdims=True)
        sel = iota == fi
        avail = avail & jnp.logical_not(sel)
        sels.append(sel)
        ms.append(m)
    m0 = ms[0]
    es = [jnp.exp(m - m0) for m in ms]
    z = es[0]
    for t in es[1:]:
        z = z + t
    aff = jnp.zeros((T, E), jnp.float32)
    for sel, t in zip(sels, es):
        aff = aff + sel.astype(jnp.float32) * (t / z)
    return aff


def _mlp_body(h_ref, logits_ref, x_ref, wg_ref, wu_ref, bg_ref, bu_ref,
              wd_ref, bd_ref, out_ref, aff_ref):
    e = pl.program_id(0)
    f = pl.program_id(1)

    @pl.when((e == 0) & (f == 0))
    def _init():
        aff_ref[...] = _topk_affinities(logits_ref[...])
        out_ref[...] = x_ref[...] + jnp.dot(
            aff_ref[...], bd_ref[...], preferred_element_type=jnp.float32)

    h = h_ref[...]
    gate = jnp.dot(h, wg_ref[0], preferred_element_type=jnp.float32) + bg_ref[0]
    up = jnp.dot(h, wu_ref[0], preferred_element_type=jnp.float32) + bu_ref[0]
    inter = up * jax.nn.sigmoid(gate)
    contrib = jnp.dot(inter, wd_ref[0], preferred_element_type=jnp.float32)
    E = aff_ref.shape[1]
    onehot = (jax.lax.broadcasted_iota(jnp.int32, (E, 1), 0) == e
              ).astype(jnp.float32)
    aff_col = jnp.dot(aff_ref[...], onehot, preferred_element_type=jnp.float32)
    out_ref[...] += aff_col * contrib


def kernel(hidden_states, rms_weight, router_weight, router_bias,
           W_gu, b_gu, W_down, b_down):
    T, H = hidden_states.shape
    E, _, I2 = W_gu.shape
    I = I2 // 2
    FB = 512
    NF = I // FB

    h, logits = pl.pallas_call(
        _rms_router_body,
        out_shape=(
            jax.ShapeDtypeStruct((T, H), jnp.float32),
            jax.ShapeDtypeStruct((T, E), jnp.float32),
        ),
    )(hidden_states, rms_weight.reshape(1, H), router_weight,
      router_bias.reshape(1, E))

    aff = _make_sc_router(T, E)(logits)

    out = pl.pallas_call(
        _mlp_body,
        grid=(E, NF),
        in_specs=[
            pl.BlockSpec((T, H), lambda e, f: (0, 0)),
            pl.BlockSpec((T, E), lambda e, f: (0, 0)),
            pl.BlockSpec((T, H), lambda e, f: (0, 0)),
            pl.BlockSpec((1, H, FB), lambda e, f: (e, 0, f)),
            pl.BlockSpec((1, H, FB), lambda e, f: (e, 0, f + NF)),
            pl.BlockSpec((1, 1, FB), lambda e, f: (e, 0, f)),
            pl.BlockSpec((1, 1, FB), lambda e, f: (e, 0, f + NF)),
            pl.BlockSpec((1, FB, H), lambda e, f: (e, f, 0)),
            pl.BlockSpec((E, H), lambda e, f: (0, 0)),
        ],
        out_specs=pl.BlockSpec((T, H), lambda e, f: (0, 0)),
        out_shape=jax.ShapeDtypeStruct((T, H), jnp.float32),
        scratch_shapes=[pltpu.VMEM((T, E), jnp.float32)],
        compiler_params=pltpu.CompilerParams(
            vmem_limit_bytes=100 * 1024 * 1024),
    )(h, logits, hidden_states, W_gu, W_gu, b_gu.reshape(E, 1, I2),
      b_gu.reshape(E, 1, I2), W_down, b_down)

    return out, aff


# speedup vs baseline: 1.0266x; 1.0005x over previous
import functools

import jax
import jax.numpy as jnp
from jax import lax
from jax.experimental import pallas as pl
from jax.experimental.pallas import tpu as pltpu
from jax.experimental.pallas import tpu_sc as plsc

_EPS = 1e-5
_TOPK = 4


def _rms_router_body(x_ref, rw_ref, w_ref, b_ref, h_ref, logits_ref):
    x = x_ref[...]
    var = jnp.mean(x * x, axis=-1, keepdims=True)
    h = x * jax.lax.rsqrt(var + _EPS) * rw_ref[...]
    logits_ref[...] = jax.lax.dot_general(
        h, w_ref[...], (((1,), (1,)), ((), ())),
        preferred_element_type=jnp.float32) + b_ref[...]
    h_ref[...] = h


def _make_sc_router(T, E):
    info = plsc.get_sparse_core_info()
    NC, NS = 1, info.num_subcores
    NW = NC * NS
    rows_per_w = T // NW
    mesh = plsc.VectorSubcoreMesh(core_axis_name="c", subcore_axis_name="s",
                                  num_cores=NC)

    @functools.partial(
        pl.kernel,
        out_type=jax.ShapeDtypeStruct((T, E), jnp.float32),
        mesh=mesh,
        scratch_types=[
            pltpu.VMEM((rows_per_w, E), jnp.float32),
            pltpu.VMEM((rows_per_w, E), jnp.float32),
        ],
        compiler_params=pltpu.CompilerParams(needs_layout_passes=False),
    )
    def _sc_router(logits_hbm, aff_hbm, rows_v, out_v):
        wid = lax.axis_index("s") * NC + lax.axis_index("c")
        base = wid * rows_per_w
        pltpu.sync_copy(logits_hbm.at[pl.ds(base, rows_per_w)], rows_v)
        iota = lax.iota(jnp.int32, E)

        def _bcast_max(v):
            return plsc.cummax(lax.rev(plsc.cummax(v), (0,)))

        for r in range(rows_per_w):
            row = rows_v[r]
            avail = jnp.ones((E,), jnp.bool_)
            sel_any = jnp.zeros((E,), jnp.bool_)
            m0 = None
            for _ in range(_TOPK):
                cur = jnp.where(avail, row, jnp.float32(-1e30))
                m = _bcast_max(cur)
                if m0 is None:
                    m0 = m
                is_m = (cur == m) & avail
                negidx = jnp.where(is_m, -iota, jnp.int32(-E))
                b = _bcast_max(negidx)
                sel = is_m & (negidx == b)
                avail = avail & jnp.logical_not(sel)
                sel_any = sel_any | sel
            expd = jnp.exp(row - m0)
            masked = jnp.where(sel_any, expd, jnp.float32(0.0))
            z = plsc.cummax(lax.rev(plsc.cumsum(masked), (0,)))
            out_v[r] = masked / z
        pltpu.sync_copy(out_v, aff_hbm.at[pl.ds(base, rows_per_w)])

    return _sc_router


def _topk_affinities(logits):
    T, E = logits.shape
    iota = jax.lax.broadcasted_iota(jnp.int32, (T, E), 1)
    avail = jnp.ones((T, E), dtype=jnp.bool_)
    neg = jnp.float32(-1e30)
    sels, ms = [], []
    for _ in range(_TOPK):
        cur = jnp.where(avail, logits, neg)
        m = jnp.max(cur, axis=-1, keepdims=True)
        is_m = (cur == m) & avail
        fi = jnp.min(jnp.where(is_m, iota, E), axis=-1, keepdims=True)
        sel = iota == fi
        avail = avail & jnp.logical_not(sel)
        sels.append(sel)
        ms.append(m)
    m0 = ms[0]
    es = [jnp.exp(m - m0) for m in ms]
    z = es[0]
    for t in es[1:]:
        z = z + t
    aff = jnp.zeros((T, E), jnp.float32)
    for sel, t in zip(sels, es):
        aff = aff + sel.astype(jnp.float32) * (t / z)
    return aff


def _mlp_body(h_ref, logits_ref, x_ref, wg_ref, wu_ref, bg_ref, bu_ref,
              wd_ref, bd_ref, out_ref, aff_ref):
    e = pl.program_id(0)
    f = pl.program_id(1)

    @pl.when((e == 0) & (f == 0))
    def _init():
        aff_ref[...] = _topk_affinities(logits_ref[...])
        out_ref[...] = x_ref[...] + jnp.dot(
            aff_ref[...], bd_ref[...], preferred_element_type=jnp.float32)

    h = h_ref[...]
    gate = jnp.dot(h, wg_ref[0], preferred_element_type=jnp.float32) + bg_ref[0]
    up = jnp.dot(h, wu_ref[0], preferred_element_type=jnp.float32) + bu_ref[0]
    inter = up * jax.nn.sigmoid(gate)
    contrib = jnp.dot(inter, wd_ref[0], preferred_element_type=jnp.float32)
    E = aff_ref.shape[1]
    onehot = (jax.lax.broadcasted_iota(jnp.int32, (E, 1), 0) == e
              ).astype(jnp.float32)
    aff_col = jnp.dot(aff_ref[...], onehot, preferred_element_type=jnp.float32)
    out_ref[...] += aff_col * contrib


def kernel(hidden_states, rms_weight, router_weight, router_bias,
           W_gu, b_gu, W_down, b_down):
    T, H = hidden_states.shape
    E, _, I2 = W_gu.shape
    I = I2 // 2
    FB = 512
    NF = I // FB

    h, logits = pl.pallas_call(
        _rms_router_body,
        out_shape=(
            jax.ShapeDtypeStruct((T, H), jnp.float32),
            jax.ShapeDtypeStruct((T, E), jnp.float32),
        ),
    )(hidden_states, rms_weight.reshape(1, H), router_weight,
      router_bias.reshape(1, E))

    out = pl.pallas_call(
        _mlp_body,
        grid=(E, NF),
        in_specs=[
            pl.BlockSpec((T, H), lambda e, f: (0, 0)),
            pl.BlockSpec((T, E), lambda e, f: (0, 0)),
            pl.BlockSpec((T, H), lambda e, f: (0, 0)),
            pl.BlockSpec((1, H, FB), lambda e, f: (e, 0, f)),
            pl.BlockSpec((1, H, FB), lambda e, f: (e, 0, f + NF)),
            pl.BlockSpec((1, 1, FB), lambda e, f: (e, 0, f)),
            pl.BlockSpec((1, 1, FB), lambda e, f: (e, 0, f + NF)),
            pl.BlockSpec((1, FB, H), lambda e, f: (e, f, 0)),
            pl.BlockSpec((E, H), lambda e, f: (0, 0)),
        ],
        out_specs=pl.BlockSpec((T, H), lambda e, f: (0, 0)),
        out_shape=jax.ShapeDtypeStruct((T, H), jnp.float32),
        scratch_shapes=[pltpu.VMEM((T, E), jnp.float32)],
        compiler_params=pltpu.CompilerParams(
            vmem_limit_bytes=100 * 1024 * 1024),
    )(h, logits, hidden_states, W_gu, W_gu, b_gu.reshape(E, 1, I2),
      b_gu.reshape(E, 1, I2), W_down, b_down)

    aff = _make_sc_router(T, E)(logits)

    return out, aff
